# baseline (speedup 1.0000x reference)
import jax
import jax.numpy as jnp
from jax import lax
from jax.experimental import pallas as pl
from jax.experimental.pallas import tpu as pltpu
from jax.experimental.pallas import tpu_sc as plsc

N_B = 4096
N_S = 50
N_D = 128

NC, NS = 2, 16
NW = NC * NS
B_PER_W = N_B // NW
NBUF = 4
GROUPS = B_PER_W // NBUF


def _emb_body(idx_hbm, w_hbm, out_hbm, idx_v, rows_v, table_sh, gsem, osem):
    sid = lax.axis_index("s")
    wid = sid * NC + lax.axis_index("c")
    base = wid * B_PER_W

    @pl.when(sid == 0)
    def _():
        pltpu.sync_copy(w_hbm, table_sh)

    pltpu.sync_copy(idx_hbm.at[wid], idx_v)
    plsc.subcore_barrier()

    def gather_desc(c, b):
        return pltpu.make_async_copy(table_sh.at[idx_v.at[c]], rows_v.at[b],
                                     gsem.at[b])

    def out_desc(c, b):
        return pltpu.make_async_copy(rows_v.at[b], out_hbm.at[base + c],
                                     osem.at[b])

    for b in range(NBUF):
        gather_desc(b, b).start()

    @pl.loop(0, GROUPS)
    def _group(g):
        c0 = g * NBUF
        for b in range(NBUF):
            gather_desc(c0 + b, b).wait()
            out_desc(c0 + b, b).start()

        @pl.when(g + 1 < GROUPS)
        def _():
            for b in range(NBUF):
                out_desc(c0 + b, b).wait()
                gather_desc(c0 + NBUF + b, b).start()

    last = (GROUPS - 1) * NBUF
    for b in range(NBUF):
        out_desc(last + b, b).wait()


_emb_call = pl.kernel(
    _emb_body,
    out_type=jax.ShapeDtypeStruct((N_B, N_S, N_D), jnp.float32),
    mesh=plsc.VectorSubcoreMesh(core_axis_name="c", subcore_axis_name="s"),
    scratch_types=[
        pltpu.VMEM((B_PER_W, N_S), jnp.int32),
        pltpu.VMEM((NBUF, N_S, N_D), jnp.float32),
        pltpu.VMEM_SHARED((1000, N_D), jnp.float32),
        pltpu.SemaphoreType.DMA((NBUF,)),
        pltpu.SemaphoreType.DMA((NBUF,)),
    ],
    compiler_params=pltpu.CompilerParams(
        use_tc_tiling_on_sc=True,
        disable_bounds_checks=True,
        disable_semaphore_checks=True,
        skip_device_barrier=True,
    ),
)


@jax.jit
def kernel(input_, weight):
    idx = input_.reshape(NW, B_PER_W, N_S)
    return _emb_call(idx, weight)


# speedup vs baseline: 2.1340x; 2.1340x over previous
import jax
import jax.numpy as jnp
from jax import lax
from jax.experimental import pallas as pl
from jax.experimental.pallas import tpu as pltpu
from jax.experimental.pallas import tpu_sc as plsc

N_B = 4096
N_S = 50
N_D = 128
N_V = 1000

NC, NS = 2, 16
NW = NC * NS
B_PER_W = N_B // NW
NBUF = 5
GROUPS = N_S // NBUF


def _emb_body(idx_hbm, w_hbm, out_hbm, idx_v, rows_v, table_sh, gsem, osem):
    sid = lax.axis_index("s")
    wid = sid * NC + lax.axis_index("c")
    base = wid * B_PER_W

    @pl.when(sid == 0)
    def _():
        pltpu.sync_copy(w_hbm, table_sh)

    pltpu.sync_copy(idx_hbm.at[:, wid], idx_v)
    plsc.subcore_barrier()

    def gather_desc(c, b):
        return pltpu.make_async_copy(table_sh.at[idx_v.at[c]], rows_v.at[b],
                                     gsem.at[b])

    def out_desc(c, b):
        dst = out_hbm.at[c, pl.ds(base, B_PER_W)]
        return pltpu.make_async_copy(rows_v.at[b], dst, osem.at[b])

    for b in range(NBUF):
        gather_desc(b, b).start()

    @pl.loop(0, GROUPS)
    def _group(g):
        c0 = g * NBUF
        for b in range(NBUF):
            gather_desc(c0 + b, b).wait()
            out_desc(c0 + b, b).start()

        @pl.when(g + 1 < GROUPS)
        def _():
            for b in range(NBUF):
                out_desc(c0 + b, b).wait()
                gather_desc(c0 + NBUF + b, b).start()

    last = (GROUPS - 1) * NBUF
    for b in range(NBUF):
        out_desc(last + b, b).wait()


_emb_call = pl.kernel(
    _emb_body,
    out_type=jax.ShapeDtypeStruct((N_S, N_B, N_D), jnp.float32),
    mesh=plsc.VectorSubcoreMesh(core_axis_name="c", subcore_axis_name="s"),
    scratch_types=[
        pltpu.VMEM((N_S, B_PER_W), jnp.int32),
        pltpu.VMEM((NBUF, B_PER_W, N_D), jnp.float32),
        pltpu.VMEM_SHARED((N_V, N_D), jnp.float32),
        pltpu.SemaphoreType.DMA((NBUF,)),
        pltpu.SemaphoreType.DMA((NBUF,)),
    ],
    compiler_params=pltpu.CompilerParams(
        use_tc_tiling_on_sc=True,
        disable_bounds_checks=True,
        disable_semaphore_checks=True,
        skip_device_barrier=True,
    ),
)


@jax.jit
def kernel(input_, weight):
    idx_t = input_.T.reshape(N_S, NW, B_PER_W)
    out_t = _emb_call(idx_t, weight)
    return out_t.transpose(1, 0, 2)


# speedup vs baseline: 2.1679x; 1.0159x over previous
import jax
import jax.numpy as jnp
from jax import lax
from jax.experimental import pallas as pl
from jax.experimental.pallas import tpu as pltpu
from jax.experimental.pallas import tpu_sc as plsc

N_B = 4096
N_S = 50
N_D = 128
N_V = 1000

NC, NS = 2, 16
NW = NC * NS
B_PER_W = N_B // NW
NBUF = 5
GROUPS = N_S // NBUF


def _emb_body(idx_hbm, w_hbm, out_hbm, idx_v, rows_v, table_sh, gsem, osem):
    sid = lax.axis_index("s")
    wid = sid * NC + lax.axis_index("c")
    base = wid * B_PER_W

    @pl.when(sid == 0)
    def _():
        pltpu.sync_copy(w_hbm, table_sh)

    pltpu.sync_copy(idx_hbm.at[:, wid], idx_v)
    plsc.subcore_barrier()

    def gather_desc(c, b):
        return pltpu.make_async_copy(table_sh.at[idx_v.at[c]], rows_v.at[b],
                                     gsem.at[b])

    def out_desc(c, b):
        dst = out_hbm.at[c, pl.ds(base, B_PER_W)]
        return pltpu.make_async_copy(rows_v.at[b], dst, osem.at[b])

    for c in range(3):
        gather_desc(c, c).start()

    for c in range(NBUF):
        gather_desc(c, c).wait()
        out_desc(c, c).start()
        if c >= 2:
            out_desc(c - 2, (c - 2) % NBUF).wait()
        gather_desc(c + 3, (c + 3) % NBUF).start()

    @pl.loop(1, GROUPS - 1)
    def _mid(g):
        c0 = g * NBUF
        for i in range(NBUF):
            gather_desc(c0 + i, i).wait()
            out_desc(c0 + i, i).start()
            out_desc(c0 + i - 2, (i - 2) % NBUF).wait()
            gather_desc(c0 + i + 3, (i + 3) % NBUF).start()

    last = (GROUPS - 1) * NBUF
    for i in range(NBUF):
        c = last + i
        gather_desc(c, i).wait()
        out_desc(c, i).start()
        out_desc(c - 2, (i - 2) % NBUF).wait()
        if c + 3 < N_S:
            gather_desc(c + 3, (i + 3) % NBUF).start()
    out_desc(N_S - 2, (NBUF - 2) % NBUF).wait()
    out_desc(N_S - 1, (NBUF - 1) % NBUF).wait()


_emb_call = pl.kernel(
    _emb_body,
    out_type=jax.ShapeDtypeStruct((N_S, N_B, N_D), jnp.float32),
    mesh=plsc.VectorSubcoreMesh(core_axis_name="c", subcore_axis_name="s"),
    scratch_types=[
        pltpu.VMEM((N_S, B_PER_W), jnp.int32),
        pltpu.VMEM((NBUF, B_PER_W, N_D), jnp.float32),
        pltpu.VMEM_SHARED((N_V, N_D), jnp.float32),
        pltpu.SemaphoreType.DMA((NBUF,)),
        pltpu.SemaphoreType.DMA((NBUF,)),
    ],
    compiler_params=pltpu.CompilerParams(
        use_tc_tiling_on_sc=True,
        disable_bounds_checks=True,
        disable_semaphore_checks=True,
        skip_device_barrier=True,
    ),
)


@jax.jit
def kernel(input_, weight):
    idx_t = input_.T.reshape(N_S, NW, B_PER_W)
    out_t = _emb_call(idx_t, weight)
    return out_t.transpose(1, 0, 2)


# speedup vs baseline: 2.1777x; 1.0045x over previous
import jax
import jax.numpy as jnp
from jax import lax
from jax.experimental import pallas as pl
from jax.experimental.pallas import tpu as pltpu
from jax.experimental.pallas import tpu_sc as plsc

N_B = 4096
N_S = 50
N_D = 128
N_V = 1000

NC, NS = 2, 16
NW = NC * NS
B_PER_W = N_B // NW
NBUF = 5
GROUPS = N_S // NBUF


def _emb_body(idx_hbm, w_hbm, out_hbm, idx_v, rows_v, table_sh, gsem, osem):
    sid = lax.axis_index("s")
    wid = sid * NC + lax.axis_index("c")
    base = wid * B_PER_W

    @pl.when(sid == 0)
    def _():
        pltpu.sync_copy(w_hbm, table_sh)

    pltpu.sync_copy(idx_hbm.at[:, pl.ds(base, B_PER_W)], idx_v)
    plsc.subcore_barrier()

    def gather_desc(c, b):
        return pltpu.make_async_copy(table_sh.at[idx_v.at[c]], rows_v.at[b],
                                     gsem.at[b])

    def out_desc(c, b):
        dst = out_hbm.at[c, pl.ds(base, B_PER_W)]
        return pltpu.make_async_copy(rows_v.at[b], dst, osem.at[b])

    for c in range(3):
        gather_desc(c, c).start()

    for c in range(NBUF):
        gather_desc(c, c).wait()
        out_desc(c, c).start()
        if c >= 2:
            out_desc(c - 2, (c - 2) % NBUF).wait()
        gather_desc(c + 3, (c + 3) % NBUF).start()

    @pl.loop(1, GROUPS - 1)
    def _mid(g):
        c0 = g * NBUF
        for i in range(NBUF):
            gather_desc(c0 + i, i).wait()
            out_desc(c0 + i, i).start()
            out_desc(c0 + i - 2, (i - 2) % NBUF).wait()
            gather_desc(c0 + i + 3, (i + 3) % NBUF).start()

    last = (GROUPS - 1) * NBUF
    for i in range(NBUF):
        c = last + i
        gather_desc(c, i).wait()
        out_desc(c, i).start()
        out_desc(c - 2, (i - 2) % NBUF).wait()
        if c + 3 < N_S:
            gather_desc(c + 3, (i + 3) % NBUF).start()
    out_desc(N_S - 2, (NBUF - 2) % NBUF).wait()
    out_desc(N_S - 1, (NBUF - 1) % NBUF).wait()


_emb_call = pl.kernel(
    _emb_body,
    out_type=jax.ShapeDtypeStruct((N_S, N_B, N_D), jnp.float32),
    mesh=plsc.VectorSubcoreMesh(core_axis_name="c", subcore_axis_name="s"),
    scratch_types=[
        pltpu.VMEM((N_S, B_PER_W), jnp.int32),
        pltpu.VMEM((NBUF, B_PER_W, N_D), jnp.float32),
        pltpu.VMEM_SHARED((N_V, N_D), jnp.float32),
        pltpu.SemaphoreType.DMA((NBUF,)),
        pltpu.SemaphoreType.DMA((NBUF,)),
    ],
    compiler_params=pltpu.CompilerParams(
        use_tc_tiling_on_sc=True,
        disable_bounds_checks=True,
        disable_semaphore_checks=True,
        skip_device_barrier=True,
    ),
)


@jax.jit
def kernel(input_, weight):
    out_t = _emb_call(input_.T, weight)
    return out_t.transpose(1, 0, 2)


# speedup vs baseline: 2.1805x; 1.0013x over previous
import jax
import jax.numpy as jnp
from jax import lax
from jax.experimental import pallas as pl
from jax.experimental.pallas import tpu as pltpu
from jax.experimental.pallas import tpu_sc as plsc

N_B = 4096
N_S = 50
N_D = 128
N_V = 1000

NC, NS = 2, 16
NW = NC * NS
B_PER_W = N_B // NW
NBUF = 5
GROUPS = N_S // NBUF


def _emb_body(idx_hbm, w_hbm, out_hbm, idx_v, rows_v, table_sh, gsem, osem):
    sid = lax.axis_index("s")
    wid = sid * NC + lax.axis_index("c")
    base = wid * B_PER_W

    @pl.when(sid == 0)
    def _():
        pltpu.sync_copy(w_hbm, table_sh)

    pltpu.sync_copy(idx_hbm.at[:, pl.ds(base, B_PER_W)], idx_v)
    plsc.subcore_barrier()

    def gather_desc(c, b):
        return pltpu.make_async_copy(table_sh.at[idx_v.at[c]], rows_v.at[b],
                                     gsem.at[b])

    def out_desc(c, b):
        dst = out_hbm.at[c, pl.ds(base, B_PER_W)]
        return pltpu.make_async_copy(rows_v.at[b], dst, osem.at[b])

    for c in range(3):
        gather_desc(c, c).start()

    for c in range(NBUF):
        gather_desc(c, c).wait()
        out_desc(c, c).start()
        if c >= 2:
            out_desc(c - 2, (c - 2) % NBUF).wait()
        gather_desc(c + 3, (c + 3) % NBUF).start()

    @pl.loop(1, GROUPS - 1)
    def _mid(g):
        c0 = g * NBUF
        for i in range(NBUF):
            gather_desc(c0 + i, i).wait()
            out_desc(c0 + i, i).start()
            out_desc(c0 + i - 2, (i - 2) % NBUF).wait()
            gather_desc(c0 + i + 3, (i + 3) % NBUF).start()

    last = (GROUPS - 1) * NBUF
    for i in range(NBUF):
        c = last + i
        gather_desc(c, i).wait()
        out_desc(c, i).start()
        out_desc(c - 2, (i - 2) % NBUF).wait()
        if c + 3 < N_S:
            gather_desc(c + 3, (i + 3) % NBUF).start()
    out_desc(N_S - 2, (NBUF - 2) % NBUF).wait()
    out_desc(N_S - 1, (NBUF - 1) % NBUF).wait()


_emb_call = pl.kernel(
    _emb_body,
    out_type=jax.ShapeDtypeStruct((N_S, N_B, N_D), jnp.float32),
    mesh=plsc.VectorSubcoreMesh(core_axis_name="c", subcore_axis_name="s"),
    scratch_types=[
        pltpu.VMEM((N_S, B_PER_W), jnp.int32),
        pltpu.VMEM((NBUF, B_PER_W, N_D), jnp.float32),
        pltpu.VMEM_SHARED((N_V, N_D), jnp.float32),
        pltpu.SemaphoreType.DMA((NBUF,)),
        pltpu.SemaphoreType.DMA((NBUF,)),
    ],
    compiler_params=pltpu.CompilerParams(use_tc_tiling_on_sc=True),
)


@jax.jit
def kernel(input_, weight):
    out_t = _emb_call(input_.T, weight)
    return out_t.transpose(1, 0, 2)
